# Initial kernel scaffold; baseline (speedup 1.0000x reference)
import jax
import jax.numpy as jnp
from jax.experimental import pallas as pl


def kernel(x, edge_index, W1, b1, W2, b2, Wl, bl):
    raise NotImplementedError("write your pallas kernel here")


# speedup vs baseline: 8.1751x; 8.1751x over previous
import functools

import jax
import jax.numpy as jnp
from jax import lax
from jax.experimental import pallas as pl
from jax.experimental.pallas import tpu as pltpu
from jax.experimental.pallas import tpu_sc as plsc

NC = 2
NS = 16
LANES = 16
EDGE_BLK = 80


def _sc_scatter_pass(tables, src, dst, n, d, gather_rows):
    e = dst.shape[0]
    e_tile = e // (NC * NS)
    nblk = e_tile // EDGE_BLK
    chunk = 400
    nchunk = n // chunk
    nsl = len(tables) if gather_rows else 1
    mesh = plsc.VectorSubcoreMesh(core_axis_name="c", subcore_axis_name="s")

    scratch = [
        pltpu.VMEM((EDGE_BLK,), jnp.int32),
        pltpu.VMEM((EDGE_BLK, d), jnp.float32),
        pltpu.VMEM((chunk, d), jnp.float32),
        pltpu.VMEM_SHARED((n, d), jnp.float32),
        pltpu.SemaphoreType.DMA,
    ]
    if gather_rows:
        scratch.insert(0, pltpu.VMEM((EDGE_BLK,), jnp.int32))

    def body(*refs):
        if gather_rows:
            tabs = refs[:nsl]
            src_h, dst_h, out_h, sidx, didx, rows, stage, acc, sem = refs[nsl:]
        else:
            dst_h, out_h, didx, rows, stage, acc, sem = refs
        cid = lax.axis_index("c")
        sid = lax.axis_index("s")
        base_e = (cid * NS + sid) * e_tile

        if not gather_rows:
            def orow(i, carry):
                rows[i, pl.ds(0, LANES)] = jnp.ones((LANES,), jnp.float32)
                return carry
            lax.fori_loop(0, EDGE_BLK, orow, 0)

        for sl in range(nsl):
            for c in range((nchunk + NS - 1) // NS):
                ci = sid + c * NS

                @pl.when(ci < nchunk)
                def _():
                    r = pl.multiple_of(ci * chunk, chunk)
                    pltpu.sync_copy(stage, acc.at[pl.ds(r, chunk)])

            plsc.subcore_barrier()

            def edge_blk(k, carry):
                off = base_e + k * EDGE_BLK
                pltpu.sync_copy(dst_h.at[pl.ds(off, EDGE_BLK)], didx)
                if gather_rows:
                    pltpu.sync_copy(src_h.at[pl.ds(off, EDGE_BLK)], sidx)
                    pltpu.async_copy(tabs[sl].at[sidx], rows, sem).wait()
                pltpu.sync_copy(rows, acc.at[didx], add=True)
                return carry
            lax.fori_loop(0, nblk, edge_blk, 0)

            plsc.subcore_barrier()

            for c in range((nchunk + NS - 1) // NS):
                ci = sid + c * NS

                @pl.when(ci < nchunk)
                def _():
                    r = pl.multiple_of(ci * chunk, chunk)
                    pltpu.sync_copy(acc.at[pl.ds(r, chunk)], stage)
                    pltpu.sync_copy(stage, out_h.at[sl, cid, pl.ds(r, chunk)])

            if sl + 1 < nsl:
                plsc.subcore_barrier()

    k = functools.partial(
        pl.kernel,
        out_type=jax.ShapeDtypeStruct((nsl, NC, n, d), jnp.float32),
        mesh=mesh,
        scratch_types=scratch,
        compiler_params=pltpu.CompilerParams(use_tc_tiling_on_sc=False),
    )(body)
    if gather_rows:
        return k(*tables, src, dst)
    return k(dst)


def _tc_matmul(x, w):
    n, din = x.shape
    dout = w.shape[1]
    blk = 400

    def body(x_ref, w_ref, o_ref):
        o_ref[...] = jnp.dot(x_ref[...], w_ref[...],
                             preferred_element_type=jnp.float32)

    return pl.pallas_call(
        body,
        grid=(n // blk,),
        in_specs=[
            pl.BlockSpec((blk, din), lambda i: (i, 0)),
            pl.BlockSpec((din, dout), lambda i: (0, 0)),
        ],
        out_specs=pl.BlockSpec((blk, dout), lambda i: (i, 0)),
        out_shape=jax.ShapeDtypeStruct((n, dout), jnp.float32),
    )(x, w)


def _dis_from_cnt(cnt_ref):
    c = cnt_ref[0, 0, :, 0:1] + cnt_ref[0, 1, :, 0:1]
    return lax.rsqrt(c + 1.0)


def _agg_from_acc(acc_ref, dis):
    return jnp.concatenate(
        [acc_ref[0, 0] + acc_ref[0, 1], acc_ref[1, 0] + acc_ref[1, 1]],
        axis=-1) * dis


def _tc_scale(cnt16, h, b):
    n, d = h.shape
    blk = 400
    hd = d // 2

    def body(cnt_ref, h_ref, b_ref, hsa_ref, hsb_ref, sc_ref):
        dis = _dis_from_cnt(cnt_ref)
        hv = h_ref[...]
        hs = hv * dis
        hsa_ref[...] = hs[:, :hd]
        hsb_ref[...] = hs[:, hd:]
        sc_ref[...] = hv * (dis * dis) + b_ref[...]

    return pl.pallas_call(
        body,
        grid=(n // blk,),
        in_specs=[
            pl.BlockSpec((1, NC, blk, 16), lambda i: (0, 0, i, 0)),
            pl.BlockSpec((blk, d), lambda i: (i, 0)),
            pl.BlockSpec((1, d), lambda i: (0, 0)),
        ],
        out_specs=[
            pl.BlockSpec((blk, hd), lambda i: (i, 0)),
            pl.BlockSpec((blk, hd), lambda i: (i, 0)),
            pl.BlockSpec((blk, d), lambda i: (i, 0)),
        ],
        out_shape=[
            jax.ShapeDtypeStruct((n, hd), jnp.float32),
            jax.ShapeDtypeStruct((n, hd), jnp.float32),
            jax.ShapeDtypeStruct((n, d), jnp.float32),
        ],
    )(cnt16, h, b)


def _tc_combine_mm(acc, cnt16, sc, w, b):
    n = sc.shape[0]
    d = sc.shape[1]
    dout = w.shape[1]
    blk = 400
    hd = dout // 2

    def body(acc_ref, cnt_ref, sc_ref, w_ref, b_ref,
             hsa_ref, hsb_ref, sc2_ref):
        dis = _dis_from_cnt(cnt_ref)
        z = jnp.maximum(_agg_from_acc(acc_ref, dis) + sc_ref[...], 0.0)
        h = jnp.dot(z, w_ref[...], preferred_element_type=jnp.float32)
        hs = h * dis
        hsa_ref[...] = hs[:, :hd]
        hsb_ref[...] = hs[:, hd:]
        sc2_ref[...] = h * (dis * dis) + b_ref[...]

    return pl.pallas_call(
        body,
        grid=(n // blk,),
        in_specs=[
            pl.BlockSpec((2, NC, blk, d // 2), lambda i: (0, 0, i, 0)),
            pl.BlockSpec((1, NC, blk, 16), lambda i: (0, 0, i, 0)),
            pl.BlockSpec((blk, d), lambda i: (i, 0)),
            pl.BlockSpec((d, dout), lambda i: (0, 0)),
            pl.BlockSpec((1, dout), lambda i: (0, 0)),
        ],
        out_specs=[
            pl.BlockSpec((blk, hd), lambda i: (i, 0)),
            pl.BlockSpec((blk, hd), lambda i: (i, 0)),
            pl.BlockSpec((blk, dout), lambda i: (i, 0)),
        ],
        out_shape=[
            jax.ShapeDtypeStruct((n, hd), jnp.float32),
            jax.ShapeDtypeStruct((n, hd), jnp.float32),
            jax.ShapeDtypeStruct((n, dout), jnp.float32),
        ],
    )(acc, cnt16, sc, w, b)


def _tc_final(acc, cnt16, sc, wl, bl):
    n = sc.shape[0]
    d = sc.shape[1]
    dout = wl.shape[1]
    blk = 400

    def body(acc_ref, cnt_ref, sc_ref, w_ref, b_ref, o_ref):
        dis = _dis_from_cnt(cnt_ref)
        z = jnp.maximum(_agg_from_acc(acc_ref, dis) + sc_ref[...], 0.0)
        o_ref[...] = jnp.dot(z, w_ref[...],
                             preferred_element_type=jnp.float32) + b_ref[...]

    return pl.pallas_call(
        body,
        grid=(n // blk,),
        in_specs=[
            pl.BlockSpec((2, NC, blk, d // 2), lambda i: (0, 0, i, 0)),
            pl.BlockSpec((1, NC, blk, 16), lambda i: (0, 0, i, 0)),
            pl.BlockSpec((blk, d), lambda i: (i, 0)),
            pl.BlockSpec((d, dout), lambda i: (0, 0)),
            pl.BlockSpec((1, dout), lambda i: (0, 0)),
        ],
        out_specs=pl.BlockSpec((blk, dout), lambda i: (i, 0)),
        out_shape=jax.ShapeDtypeStruct((n, dout), jnp.float32),
    )(acc, cnt16, sc, wl, bl)


def kernel(x, edge_index, W1, b1, W2, b2, Wl, bl):
    n = x.shape[0]
    src = edge_index[0]
    dst = edge_index[1]
    b1r = b1.reshape(1, -1)
    b2r = b2.reshape(1, -1)
    blr = bl.reshape(1, -1)

    cnt16 = _sc_scatter_pass(None, None, dst, n, 16, gather_rows=False)
    h1 = _tc_matmul(x, W1)
    hsa, hsb, sc1 = _tc_scale(cnt16, h1, b1r)

    acc1 = _sc_scatter_pass((hsa, hsb), src, dst, n, hsa.shape[1],
                            gather_rows=True)
    hs2a, hs2b, sc2 = _tc_combine_mm(acc1, cnt16, sc1, W2, b2r)

    acc2 = _sc_scatter_pass((hs2a, hs2b), src, dst, n, hs2a.shape[1],
                            gather_rows=True)
    return _tc_final(acc2, cnt16, sc2, Wl, blr)


# speedup vs baseline: 26.0032x; 3.1808x over previous
import functools

import jax
import jax.numpy as jnp
from jax import lax
from jax.experimental import pallas as pl
from jax.experimental.pallas import tpu as pltpu
from jax.experimental.pallas import tpu_sc as plsc

NC = 2
NS = 16
LANES = 16
EDGE_BLK = 80
NBUF = 5
CHUNK = 400


def _fill(buf, nrows, d, val):
    nv = d // LANES

    def row(i, c):
        for j in range(nv):
            buf[i, pl.ds(j * LANES, LANES)] = jnp.full((LANES,), val,
                                                       jnp.float32)
        return c
    lax.fori_loop(0, nrows, row, 0)


def _zero_chunks(zbuf, acc, sid, nchunk, chunk):
    for c in range((nchunk + NS - 1) // NS):
        ci = sid + c * NS

        @pl.when(ci < nchunk)
        def _():
            r = pl.multiple_of(ci * chunk, chunk)
            pltpu.sync_copy(zbuf, acc.at[pl.ds(r, chunk)])


def _dump_chunks(acc, stage, out2d, sid, nchunk, chunk):
    for c in range((nchunk + NS - 1) // NS):
        ci = sid + c * NS

        @pl.when(ci < nchunk)
        def _():
            r = pl.multiple_of(ci * chunk, chunk)
            pltpu.sync_copy(acc.at[pl.ds(r, chunk)], stage)
            pltpu.sync_copy(stage, out2d.at[pl.ds(r, chunk)])


def _sc_degree_pass(dst2, n):
    d = LANES
    nblk = dst2.shape[0] // (NC * NS)
    nchunk = n // CHUNK
    mesh = plsc.VectorSubcoreMesh(core_axis_name="c", subcore_axis_name="s")

    def body(dst_h, out_h, didx, ones, zbuf, stage, acc):
        cid = lax.axis_index("c")
        sid = lax.axis_index("s")
        tile = cid * NS + sid

        pltpu.sync_copy(dst_h.at[pl.ds(tile * nblk, nblk)], didx)
        _fill(ones, EDGE_BLK, d, 1.0)
        _fill(zbuf, CHUNK, d, 0.0)
        _zero_chunks(zbuf, acc, sid, nchunk, CHUNK)
        plsc.subcore_barrier()

        def blk(k, c):
            pltpu.sync_copy(ones, acc.at[didx.at[k]], add=True)
            return c
        lax.fori_loop(0, nblk, blk, 0)

        plsc.subcore_barrier()
        _dump_chunks(acc, stage, out_h.at[cid], sid, nchunk, CHUNK)

    return pl.kernel(
        body,
        out_type=jax.ShapeDtypeStruct((NC, n, d), jnp.float32),
        mesh=mesh,
        scratch_types=[
            pltpu.VMEM((nblk, EDGE_BLK), jnp.int32),
            pltpu.VMEM((EDGE_BLK, d), jnp.float32),
            pltpu.VMEM((CHUNK, d), jnp.float32),
            pltpu.VMEM((CHUNK, d), jnp.float32),
            pltpu.VMEM_SHARED((n, d), jnp.float32),
        ],
        compiler_params=pltpu.CompilerParams(use_tc_tiling_on_sc=False),
    )(dst2)


def _sc_gather_pass(tables, src2, dst2, n, d):
    nblk = dst2.shape[0] // (NC * NS)
    chunk = EDGE_BLK
    nchunk = n // chunk
    nsl = len(tables)
    mesh = plsc.VectorSubcoreMesh(core_axis_name="c", subcore_axis_name="s")

    def body(*refs):
        tabs = refs[:nsl]
        (src_h, dst_h, out_h, sidx, didx) = refs[nsl:nsl + 5]
        rows = refs[nsl + 5:nsl + 5 + NBUF]
        acc = refs[nsl + 5 + NBUF]
        gsem = refs[nsl + 6 + NBUF:nsl + 6 + 2 * NBUF]
        ssem = refs[nsl + 6 + 2 * NBUF]
        cid = lax.axis_index("c")
        sid = lax.axis_index("s")
        tile = cid * NS + sid

        pltpu.sync_copy(src_h.at[pl.ds(tile * nblk, nblk)], sidx)
        pltpu.sync_copy(dst_h.at[pl.ds(tile * nblk, nblk)], didx)

        for sl in range(nsl):
            _fill(rows[0], chunk, d, 0.0)
            _zero_chunks(rows[0], acc, sid, nchunk, chunk)
            plsc.subcore_barrier()

            for b in range(NBUF - 1):
                pltpu.async_copy(tabs[sl].at[sidx.at[b]], rows[b], gsem[b])

            def outer(i, c):
                h = [None]
                for b in range(NBUF):
                    blk = i * NBUF + b
                    nb = (b + NBUF - 1) % NBUF
                    if h[0] is not None:
                        h[0].wait()

                    @pl.when(blk + NBUF - 1 < nblk)
                    def _():
                        pltpu.async_copy(
                            tabs[sl].at[sidx.at[blk + NBUF - 1]],
                            rows[nb], gsem[nb])
                    pltpu.make_async_copy(
                        tabs[sl].at[sidx.at[blk]], rows[b], gsem[b]).wait()
                    h[0] = pltpu.async_copy(rows[b], acc.at[didx.at[blk]],
                                            ssem, add=True)
                h[0].wait()
                return c
            lax.fori_loop(0, nblk // NBUF, outer, 0)

            plsc.subcore_barrier()
            _dump_chunks(acc, rows[0], out_h.at[sl, cid], sid, nchunk, chunk)
            if sl + 1 < nsl:
                plsc.subcore_barrier()

    scratch = [
        pltpu.VMEM((nblk, EDGE_BLK), jnp.int32),
        pltpu.VMEM((nblk, EDGE_BLK), jnp.int32),
    ]
    scratch += [pltpu.VMEM((EDGE_BLK, d), jnp.float32) for _ in range(NBUF)]
    scratch += [pltpu.VMEM_SHARED((n, d), jnp.float32)]
    scratch += [pltpu.SemaphoreType.DMA for _ in range(NBUF + 1)]

    return pl.kernel(
        body,
        out_type=jax.ShapeDtypeStruct((nsl, NC, n, d), jnp.float32),
        mesh=mesh,
        scratch_types=scratch,
        compiler_params=pltpu.CompilerParams(use_tc_tiling_on_sc=False),
    )(*tables, src2, dst2)


def _tc_matmul(x, w):
    n, din = x.shape
    dout = w.shape[1]
    blk = 400

    def body(x_ref, w_ref, o_ref):
        o_ref[...] = jnp.dot(x_ref[...], w_ref[...],
                             preferred_element_type=jnp.float32)

    return pl.pallas_call(
        body,
        grid=(n // blk,),
        in_specs=[
            pl.BlockSpec((blk, din), lambda i: (i, 0)),
            pl.BlockSpec((din, dout), lambda i: (0, 0)),
        ],
        out_specs=pl.BlockSpec((blk, dout), lambda i: (i, 0)),
        out_shape=jax.ShapeDtypeStruct((n, dout), jnp.float32),
    )(x, w)


def _dis_from_cnt(cnt_ref):
    c = cnt_ref[0, :, 0:1] + cnt_ref[1, :, 0:1]
    return lax.rsqrt(c + 1.0)


def _agg_from_acc(acc_ref, dis):
    return jnp.concatenate(
        [acc_ref[0, 0] + acc_ref[0, 1], acc_ref[1, 0] + acc_ref[1, 1]],
        axis=-1) * dis


def _tc_scale(cnt16, h, b):
    n, d = h.shape
    blk = 400
    hd = d // 2

    def body(cnt_ref, h_ref, b_ref, hsa_ref, hsb_ref, sc_ref):
        dis = _dis_from_cnt(cnt_ref)
        hv = h_ref[...]
        hs = hv * dis
        hsa_ref[...] = hs[:, :hd]
        hsb_ref[...] = hs[:, hd:]
        sc_ref[...] = hv * (dis * dis) + b_ref[...]

    return pl.pallas_call(
        body,
        grid=(n // blk,),
        in_specs=[
            pl.BlockSpec((NC, blk, 16), lambda i: (0, i, 0)),
            pl.BlockSpec((blk, d), lambda i: (i, 0)),
            pl.BlockSpec((1, d), lambda i: (0, 0)),
        ],
        out_specs=[
            pl.BlockSpec((blk, hd), lambda i: (i, 0)),
            pl.BlockSpec((blk, hd), lambda i: (i, 0)),
            pl.BlockSpec((blk, d), lambda i: (i, 0)),
        ],
        out_shape=[
            jax.ShapeDtypeStruct((n, hd), jnp.float32),
            jax.ShapeDtypeStruct((n, hd), jnp.float32),
            jax.ShapeDtypeStruct((n, d), jnp.float32),
        ],
    )(cnt16, h, b)


def _tc_combine_mm(acc, cnt16, sc, w, b):
    n = sc.shape[0]
    d = sc.shape[1]
    dout = w.shape[1]
    blk = 400
    hd = dout // 2

    def body(acc_ref, cnt_ref, sc_ref, w_ref, b_ref,
             hsa_ref, hsb_ref, sc2_ref):
        dis = _dis_from_cnt(cnt_ref)
        z = jnp.maximum(_agg_from_acc(acc_ref, dis) + sc_ref[...], 0.0)
        h = jnp.dot(z, w_ref[...], preferred_element_type=jnp.float32)
        hs = h * dis
        hsa_ref[...] = hs[:, :hd]
        hsb_ref[...] = hs[:, hd:]
        sc2_ref[...] = h * (dis * dis) + b_ref[...]

    return pl.pallas_call(
        body,
        grid=(n // blk,),
        in_specs=[
            pl.BlockSpec((2, NC, blk, d // 2), lambda i: (0, 0, i, 0)),
            pl.BlockSpec((NC, blk, 16), lambda i: (0, i, 0)),
            pl.BlockSpec((blk, d), lambda i: (i, 0)),
            pl.BlockSpec((d, dout), lambda i: (0, 0)),
            pl.BlockSpec((1, dout), lambda i: (0, 0)),
        ],
        out_specs=[
            pl.BlockSpec((blk, hd), lambda i: (i, 0)),
            pl.BlockSpec((blk, hd), lambda i: (i, 0)),
            pl.BlockSpec((blk, dout), lambda i: (i, 0)),
        ],
        out_shape=[
            jax.ShapeDtypeStruct((n, hd), jnp.float32),
            jax.ShapeDtypeStruct((n, hd), jnp.float32),
            jax.ShapeDtypeStruct((n, dout), jnp.float32),
        ],
    )(acc, cnt16, sc, w, b)


def _tc_final(acc, cnt16, sc, wl, bl):
    n = sc.shape[0]
    d = sc.shape[1]
    dout = wl.shape[1]
    blk = 400

    def body(acc_ref, cnt_ref, sc_ref, w_ref, b_ref, o_ref):
        dis = _dis_from_cnt(cnt_ref)
        z = jnp.maximum(_agg_from_acc(acc_ref, dis) + sc_ref[...], 0.0)
        o_ref[...] = jnp.dot(z, w_ref[...],
                             preferred_element_type=jnp.float32) + b_ref[...]

    return pl.pallas_call(
        body,
        grid=(n // blk,),
        in_specs=[
            pl.BlockSpec((2, NC, blk, d // 2), lambda i: (0, 0, i, 0)),
            pl.BlockSpec((NC, blk, 16), lambda i: (0, i, 0)),
            pl.BlockSpec((blk, d), lambda i: (i, 0)),
            pl.BlockSpec((d, dout), lambda i: (0, 0)),
            pl.BlockSpec((1, dout), lambda i: (0, 0)),
        ],
        out_specs=pl.BlockSpec((blk, dout), lambda i: (i, 0)),
        out_shape=jax.ShapeDtypeStruct((n, dout), jnp.float32),
    )(acc, cnt16, sc, wl, bl)


def kernel(x, edge_index, W1, b1, W2, b2, Wl, bl):
    n = x.shape[0]
    src2 = edge_index[0].reshape(-1, EDGE_BLK)
    dst2 = edge_index[1].reshape(-1, EDGE_BLK)
    b1r = b1.reshape(1, -1)
    b2r = b2.reshape(1, -1)
    blr = bl.reshape(1, -1)

    cnt16 = _sc_degree_pass(dst2, n)
    h1 = _tc_matmul(x, W1)
    hsa, hsb, sc1 = _tc_scale(cnt16, h1, b1r)

    acc1 = _sc_gather_pass((hsa, hsb), src2, dst2, n, hsa.shape[1])
    hs2a, hs2b, sc2 = _tc_combine_mm(acc1, cnt16, sc1, W2, b2r)

    acc2 = _sc_gather_pass((hs2a, hs2b), src2, dst2, n, hs2a.shape[1])
    return _tc_final(acc2, cnt16, sc2, Wl, blr)


# speedup vs baseline: 31.1454x; 1.1978x over previous
import jax
import jax.numpy as jnp
from jax import lax
from jax.experimental import pallas as pl
from jax.experimental.pallas import tpu as pltpu
from jax.experimental.pallas import tpu_sc as plsc

NC = 2
NS = 16
LANES = 16
EDGE_BLK = 80
NBUF = 3


def _fill(buf, nrows, d, val):
    nv = d // LANES

    def row(i, c):
        for j in range(nv):
            buf[i, pl.ds(j * LANES, LANES)] = jnp.full((LANES,), val,
                                                       jnp.float32)
        return c
    lax.fori_loop(0, nrows, row, 0)


def _zero_chunks(zbuf, acc, sid, nchunk, chunk):
    for c in range((nchunk + NS - 1) // NS):
        ci = sid + c * NS

        @pl.when(ci < nchunk)
        def _():
            r = pl.multiple_of(ci * chunk, chunk)
            pltpu.sync_copy(zbuf, acc.at[pl.ds(r, chunk)])


def _dump_chunks(acc, stage, out2d, sid, nchunk, chunk):
    for c in range((nchunk + NS - 1) // NS):
        ci = sid + c * NS

        @pl.when(ci < nchunk)
        def _():
            r = pl.multiple_of(ci * chunk, chunk)
            pltpu.sync_copy(acc.at[pl.ds(r, chunk)], stage)
            pltpu.sync_copy(stage, out2d.at[pl.ds(r, chunk)])


def _sc_degree_pass(dst2, n):
    d = LANES
    chunk = 400
    nblk = dst2.shape[0] // (NC * NS)
    nchunk = n // chunk
    mesh = plsc.VectorSubcoreMesh(core_axis_name="c", subcore_axis_name="s")

    def body(dst_h, out_h, didx, ones, zbuf, stage, acc):
        cid = lax.axis_index("c")
        sid = lax.axis_index("s")
        tile = cid * NS + sid

        pltpu.sync_copy(dst_h.at[pl.ds(tile * nblk, nblk)], didx)
        _fill(ones, EDGE_BLK, d, 1.0)
        _fill(zbuf, chunk, d, 0.0)
        _zero_chunks(zbuf, acc, sid, nchunk, chunk)
        plsc.subcore_barrier()

        def blk(k, c):
            pltpu.sync_copy(ones, acc.at[didx.at[k]], add=True)
            return c
        lax.fori_loop(0, nblk, blk, 0)

        plsc.subcore_barrier()
        _dump_chunks(acc, stage, out_h.at[cid], sid, nchunk, chunk)

    return pl.kernel(
        body,
        out_type=jax.ShapeDtypeStruct((NC, n, d), jnp.float32),
        mesh=mesh,
        scratch_types=[
            pltpu.VMEM((nblk, EDGE_BLK), jnp.int32),
            pltpu.VMEM((EDGE_BLK, d), jnp.float32),
            pltpu.VMEM((chunk, d), jnp.float32),
            pltpu.VMEM((chunk, d), jnp.float32),
            pltpu.VMEM_SHARED((n, d), jnp.float32),
        ],
        compiler_params=pltpu.CompilerParams(use_tc_tiling_on_sc=False),
    )(dst2)


def _sc_gather_pass(table, src2, dst2, n, d):
    nblk = dst2.shape[0] // (NC * NS)
    chunk = EDGE_BLK
    nchunk = n // chunk
    nmain = (nblk // NBUF) * NBUF
    mesh = plsc.VectorSubcoreMesh(core_axis_name="c", subcore_axis_name="s")

    def body(*refs):
        (tab, src_h, dst_h, out_h, sidx, didx) = refs[:6]
        rows = refs[6:6 + NBUF]
        acc = refs[6 + NBUF]
        gsem = refs[7 + NBUF:7 + 2 * NBUF]
        ssem = refs[7 + 2 * NBUF]
        cid = lax.axis_index("c")
        sid = lax.axis_index("s")
        tile = cid * NS + sid

        pltpu.sync_copy(src_h.at[pl.ds(tile * nblk, nblk)], sidx)
        pltpu.sync_copy(dst_h.at[pl.ds(tile * nblk, nblk)], didx)

        _fill(rows[0], chunk, d, 0.0)
        _zero_chunks(rows[0], acc, sid, nchunk, chunk)
        plsc.subcore_barrier()

        for b in range(NBUF - 1):
            pltpu.async_copy(tab.at[sidx.at[b]], rows[b], gsem[b])

        def outer(i, c):
            h = [None]
            for b in range(NBUF):
                blk = i * NBUF + b
                nb = (b + NBUF - 1) % NBUF
                if h[0] is not None:
                    h[0].wait()

                @pl.when(blk + NBUF - 1 < nblk)
                def _():
                    pltpu.async_copy(tab.at[sidx.at[blk + NBUF - 1]],
                                     rows[nb], gsem[nb])
                pltpu.make_async_copy(tab.at[sidx.at[blk]], rows[b],
                                      gsem[b]).wait()
                h[0] = pltpu.async_copy(rows[b], acc.at[didx.at[blk]],
                                        ssem, add=True)
            h[0].wait()
            return c
        lax.fori_loop(0, nblk // NBUF, outer, 0)

        for t in range(nblk - nmain):
            blk = nmain + t
            b = blk % NBUF
            pltpu.make_async_copy(tab.at[sidx.at[blk]], rows[b],
                                  gsem[b]).wait()
            pltpu.sync_copy(rows[b], acc.at[didx.at[blk]], add=True)

        plsc.subcore_barrier()
        _dump_chunks(acc, rows[0], out_h.at[cid], sid, nchunk, chunk)

    scratch = [
        pltpu.VMEM((nblk, EDGE_BLK), jnp.int32),
        pltpu.VMEM((nblk, EDGE_BLK), jnp.int32),
    ]
    scratch += [pltpu.VMEM((EDGE_BLK, d), jnp.float32) for _ in range(NBUF)]
    scratch += [pltpu.VMEM_SHARED((n, d), jnp.float32)]
    scratch += [pltpu.SemaphoreType.DMA for _ in range(NBUF + 1)]

    return pl.kernel(
        body,
        out_type=jax.ShapeDtypeStruct((NC, n, d), jnp.float32),
        mesh=mesh,
        scratch_types=scratch,
        compiler_params=pltpu.CompilerParams(use_tc_tiling_on_sc=False),
    )(table, src2, dst2)


def _dis_from_cnt(cnt_ref):
    c = cnt_ref[0, :, 0:1] + cnt_ref[1, :, 0:1]
    return lax.rsqrt(c + 1.0)


def _tc_mm_scale(cnt16, x, w, b):
    n, din = x.shape
    dout = w.shape[1]
    blk = 400

    def body(cnt_ref, x_ref, w_ref, b_ref, hs_ref, sc_ref):
        dis = _dis_from_cnt(cnt_ref)
        h = jnp.dot(x_ref[...], w_ref[...],
                    preferred_element_type=jnp.float32)
        hs_ref[...] = h * dis
        sc_ref[...] = h * (dis * dis) + b_ref[...]

    return pl.pallas_call(
        body,
        grid=(n // blk,),
        in_specs=[
            pl.BlockSpec((NC, blk, 16), lambda i: (0, i, 0)),
            pl.BlockSpec((blk, din), lambda i: (i, 0)),
            pl.BlockSpec((din, dout), lambda i: (0, 0)),
            pl.BlockSpec((1, dout), lambda i: (0, 0)),
        ],
        out_specs=[
            pl.BlockSpec((blk, dout), lambda i: (i, 0)),
            pl.BlockSpec((blk, dout), lambda i: (i, 0)),
        ],
        out_shape=[
            jax.ShapeDtypeStruct((n, dout), jnp.float32),
            jax.ShapeDtypeStruct((n, dout), jnp.float32),
        ],
    )(cnt16, x, w, b)


def _tc_combine_mm(acc, cnt16, sc, w, b):
    n = sc.shape[0]
    d = sc.shape[1]
    dout = w.shape[1]
    blk = 400

    def body(acc_ref, cnt_ref, sc_ref, w_ref, b_ref, hs_ref, sc2_ref):
        dis = _dis_from_cnt(cnt_ref)
        agg = (acc_ref[0] + acc_ref[1]) * dis
        z = jnp.maximum(agg + sc_ref[...], 0.0)
        h = jnp.dot(z, w_ref[...], preferred_element_type=jnp.float32)
        hs_ref[...] = h * dis
        sc2_ref[...] = h * (dis * dis) + b_ref[...]

    return pl.pallas_call(
        body,
        grid=(n // blk,),
        in_specs=[
            pl.BlockSpec((NC, blk, d), lambda i: (0, i, 0)),
            pl.BlockSpec((NC, blk, 16), lambda i: (0, i, 0)),
            pl.BlockSpec((blk, d), lambda i: (i, 0)),
            pl.BlockSpec((d, dout), lambda i: (0, 0)),
            pl.BlockSpec((1, dout), lambda i: (0, 0)),
        ],
        out_specs=[
            pl.BlockSpec((blk, dout), lambda i: (i, 0)),
            pl.BlockSpec((blk, dout), lambda i: (i, 0)),
        ],
        out_shape=[
            jax.ShapeDtypeStruct((n, dout), jnp.float32),
            jax.ShapeDtypeStruct((n, dout), jnp.float32),
        ],
    )(acc, cnt16, sc, w, b)


def _tc_final(acc, cnt16, sc, wl, bl):
    n = sc.shape[0]
    d = sc.shape[1]
    dout = wl.shape[1]
    blk = 400

    def body(acc_ref, cnt_ref, sc_ref, w_ref, b_ref, o_ref):
        dis = _dis_from_cnt(cnt_ref)
        agg = (acc_ref[0] + acc_ref[1]) * dis
        z = jnp.maximum(agg + sc_ref[...], 0.0)
        o_ref[...] = jnp.dot(z, w_ref[...],
                             preferred_element_type=jnp.float32) + b_ref[...]

    return pl.pallas_call(
        body,
        grid=(n // blk,),
        in_specs=[
            pl.BlockSpec((NC, blk, d), lambda i: (0, i, 0)),
            pl.BlockSpec((NC, blk, 16), lambda i: (0, i, 0)),
            pl.BlockSpec((blk, d), lambda i: (i, 0)),
            pl.BlockSpec((d, dout), lambda i: (0, 0)),
            pl.BlockSpec((1, dout), lambda i: (0, 0)),
        ],
        out_specs=pl.BlockSpec((blk, dout), lambda i: (i, 0)),
        out_shape=jax.ShapeDtypeStruct((n, dout), jnp.float32),
    )(acc, cnt16, sc, wl, bl)


def kernel(x, edge_index, W1, b1, W2, b2, Wl, bl):
    n = x.shape[0]
    src2 = edge_index[0].reshape(-1, EDGE_BLK)
    dst2 = edge_index[1].reshape(-1, EDGE_BLK)
    b1r = b1.reshape(1, -1)
    b2r = b2.reshape(1, -1)
    blr = bl.reshape(1, -1)

    cnt16 = _sc_degree_pass(dst2, n)
    hs1, sc1 = _tc_mm_scale(cnt16, x, W1, b1r)

    acc1 = _sc_gather_pass(hs1, src2, dst2, n, hs1.shape[1])
    hs2, sc2 = _tc_combine_mm(acc1, cnt16, sc1, W2, b2r)

    acc2 = _sc_gather_pass(hs2, src2, dst2, n, hs2.shape[1])
    return _tc_final(acc2, cnt16, sc2, Wl, blr)


# speedup vs baseline: 32.0083x; 1.0277x over previous
import jax
import jax.numpy as jnp
from jax import lax
from jax.experimental import pallas as pl
from jax.experimental.pallas import tpu as pltpu
from jax.experimental.pallas import tpu_sc as plsc

NC = 2
NS = 16
LANES = 16
EDGE_BLK = 80
NBUF = 3


def _fill(buf, nrows, d, val):
    nv = d // LANES

    def row(i, c):
        for j in range(nv):
            buf[i, pl.ds(j * LANES, LANES)] = jnp.full((LANES,), val,
                                                       jnp.float32)
        return c
    lax.fori_loop(0, nrows, row, 0)


def _zero_chunks(zbuf, acc, sid, nchunk, chunk):
    for c in range((nchunk + NS - 1) // NS):
        ci = sid + c * NS

        @pl.when(ci < nchunk)
        def _():
            r = pl.multiple_of(ci * chunk, chunk)
            pltpu.sync_copy(zbuf, acc.at[pl.ds(r, chunk)])


def _dump_chunks(acc, out2d, sid, nchunk, chunk):
    for c in range((nchunk + NS - 1) // NS):
        ci = sid + c * NS

        @pl.when(ci < nchunk)
        def _():
            r = pl.multiple_of(ci * chunk, chunk)
            pltpu.sync_copy(acc.at[pl.ds(r, chunk)], out2d.at[pl.ds(r, chunk)])


def _sc_degree_pass(dst2, n):
    d = LANES
    chunk = 400
    nblk = dst2.shape[0] // (NC * NS)
    nchunk = n // chunk
    mesh = plsc.VectorSubcoreMesh(core_axis_name="c", subcore_axis_name="s")

    def body(dst_h, out_h, didx, ones, zbuf, acc, isem):
        cid = lax.axis_index("c")
        sid = lax.axis_index("s")
        tile = cid * NS + sid

        ih = pltpu.async_copy(dst_h.at[pl.ds(tile * nblk, nblk)], didx, isem)
        _fill(ones, EDGE_BLK, d, 1.0)
        _fill(zbuf, chunk, d, 0.0)
        _zero_chunks(zbuf, acc, sid, nchunk, chunk)
        ih.wait()
        plsc.subcore_barrier()

        def blk(k, c):
            pltpu.sync_copy(ones, acc.at[didx.at[k]], add=True)
            return c
        lax.fori_loop(0, nblk, blk, 0)

        plsc.subcore_barrier()
        _dump_chunks(acc, out_h.at[cid], sid, nchunk, chunk)

    return pl.kernel(
        body,
        out_type=jax.ShapeDtypeStruct((NC, n, d), jnp.float32),
        mesh=mesh,
        scratch_types=[
            pltpu.VMEM((nblk, EDGE_BLK), jnp.int32),
            pltpu.VMEM((EDGE_BLK, d), jnp.float32),
            pltpu.VMEM((chunk, d), jnp.float32),
            pltpu.VMEM_SHARED((n, d), jnp.float32),
            pltpu.SemaphoreType.DMA,
        ],
        compiler_params=pltpu.CompilerParams(use_tc_tiling_on_sc=False),
    )(dst2)


def _sc_gather_pass(table, src2, dst2, n, d):
    nblk = dst2.shape[0] // (NC * NS)
    chunk = EDGE_BLK
    nchunk = n // chunk
    nmain = (nblk // NBUF) * NBUF
    mesh = plsc.VectorSubcoreMesh(core_axis_name="c", subcore_axis_name="s")

    def body(*refs):
        (tab, src_h, dst_h, out_h, sidx, didx) = refs[:6]
        rows = refs[6:6 + NBUF]
        acc = refs[6 + NBUF]
        gsem = refs[7 + NBUF:7 + 2 * NBUF]
        ssem = refs[7 + 2 * NBUF]
        isem = refs[8 + 2 * NBUF]
        cid = lax.axis_index("c")
        sid = lax.axis_index("s")
        tile = cid * NS + sid

        ih1 = pltpu.async_copy(src_h.at[pl.ds(tile * nblk, nblk)], sidx, isem)
        ih2 = pltpu.async_copy(dst_h.at[pl.ds(tile * nblk, nblk)], didx, isem)

        _fill(rows[0], chunk, d, 0.0)
        _zero_chunks(rows[0], acc, sid, nchunk, chunk)
        ih1.wait()
        ih2.wait()
        plsc.subcore_barrier()

        for b in range(NBUF - 1):
            pltpu.async_copy(tab.at[sidx.at[b]], rows[b], gsem[b])

        def outer(i, c):
            h = [None]
            for b in range(NBUF):
                blk = i * NBUF + b
                nb = (b + NBUF - 1) % NBUF
                if h[0] is not None:
                    h[0].wait()

                @pl.when(blk + NBUF - 1 < nblk)
                def _():
                    pltpu.async_copy(tab.at[sidx.at[blk + NBUF - 1]],
                                     rows[nb], gsem[nb])
                pltpu.make_async_copy(tab.at[sidx.at[blk]], rows[b],
                                      gsem[b]).wait()
                h[0] = pltpu.async_copy(rows[b], acc.at[didx.at[blk]],
                                        ssem, add=True)
            h[0].wait()
            return c
        lax.fori_loop(0, nblk // NBUF, outer, 0)

        for t in range(nblk - nmain):
            blk = nmain + t
            b = blk % NBUF
            pltpu.make_async_copy(tab.at[sidx.at[blk]], rows[b],
                                  gsem[b]).wait()
            pltpu.sync_copy(rows[b], acc.at[didx.at[blk]], add=True)

        plsc.subcore_barrier()
        _dump_chunks(acc, out_h.at[cid], sid, nchunk, chunk)

    scratch = [
        pltpu.VMEM((nblk, EDGE_BLK), jnp.int32),
        pltpu.VMEM((nblk, EDGE_BLK), jnp.int32),
    ]
    scratch += [pltpu.VMEM((EDGE_BLK, d), jnp.float32) for _ in range(NBUF)]
    scratch += [pltpu.VMEM_SHARED((n, d), jnp.float32)]
    scratch += [pltpu.SemaphoreType.DMA for _ in range(NBUF + 2)]

    return pl.kernel(
        body,
        out_type=jax.ShapeDtypeStruct((NC, n, d), jnp.float32),
        mesh=mesh,
        scratch_types=scratch,
        compiler_params=pltpu.CompilerParams(use_tc_tiling_on_sc=False),
    )(table, src2, dst2)


def _dis_from_cnt(cnt_ref):
    c = cnt_ref[0, :, 0:1] + cnt_ref[1, :, 0:1]
    return lax.rsqrt(c + 1.0)


def _tc_mm(x, w):
    n, din = x.shape
    dout = w.shape[1]
    blk = 400

    def body(x_ref, w_ref, h_ref):
        h_ref[...] = jnp.dot(x_ref[...], w_ref[...],
                             preferred_element_type=jnp.float32)

    return pl.pallas_call(
        body,
        grid=(n // blk,),
        in_specs=[
            pl.BlockSpec((blk, din), lambda i: (i, 0)),
            pl.BlockSpec((din, dout), lambda i: (0, 0)),
        ],
        out_specs=pl.BlockSpec((blk, dout), lambda i: (i, 0)),
        out_shape=jax.ShapeDtypeStruct((n, dout), jnp.float32),
    )(x, w)


def _tc_scale(cnt16, h, b):
    n, dout = h.shape
    blk = 400

    def body(cnt_ref, h_ref, b_ref, hs_ref, sc_ref):
        dis = _dis_from_cnt(cnt_ref)
        h = h_ref[...]
        hs_ref[...] = h * dis
        sc_ref[...] = h * (dis * dis) + b_ref[...]

    return pl.pallas_call(
        body,
        grid=(n // blk,),
        in_specs=[
            pl.BlockSpec((NC, blk, 16), lambda i: (0, i, 0)),
            pl.BlockSpec((blk, dout), lambda i: (i, 0)),
            pl.BlockSpec((1, dout), lambda i: (0, 0)),
        ],
        out_specs=[
            pl.BlockSpec((blk, dout), lambda i: (i, 0)),
            pl.BlockSpec((blk, dout), lambda i: (i, 0)),
        ],
        out_shape=[
            jax.ShapeDtypeStruct((n, dout), jnp.float32),
            jax.ShapeDtypeStruct((n, dout), jnp.float32),
        ],
    )(cnt16, h, b)


def _tc_combine_mm(acc, cnt16, sc, w, b):
    n = sc.shape[0]
    d = sc.shape[1]
    dout = w.shape[1]
    blk = 400

    def body(acc_ref, cnt_ref, sc_ref, w_ref, b_ref, hs_ref, sc2_ref):
        dis = _dis_from_cnt(cnt_ref)
        agg = (acc_ref[0] + acc_ref[1]) * dis
        z = jnp.maximum(agg + sc_ref[...], 0.0)
        h = jnp.dot(z, w_ref[...], preferred_element_type=jnp.float32)
        hs_ref[...] = h * dis
        sc2_ref[...] = h * (dis * dis) + b_ref[...]

    return pl.pallas_call(
        body,
        grid=(n // blk,),
        in_specs=[
            pl.BlockSpec((NC, blk, d), lambda i: (0, i, 0)),
            pl.BlockSpec((NC, blk, 16), lambda i: (0, i, 0)),
            pl.BlockSpec((blk, d), lambda i: (i, 0)),
            pl.BlockSpec((d, dout), lambda i: (0, 0)),
            pl.BlockSpec((1, dout), lambda i: (0, 0)),
        ],
        out_specs=[
            pl.BlockSpec((blk, dout), lambda i: (i, 0)),
            pl.BlockSpec((blk, dout), lambda i: (i, 0)),
        ],
        out_shape=[
            jax.ShapeDtypeStruct((n, dout), jnp.float32),
            jax.ShapeDtypeStruct((n, dout), jnp.float32),
        ],
    )(acc, cnt16, sc, w, b)


def _tc_final(acc, cnt16, sc, wl, bl):
    n = sc.shape[0]
    d = sc.shape[1]
    dout = wl.shape[1]
    blk = 400

    def body(acc_ref, cnt_ref, sc_ref, w_ref, b_ref, o_ref):
        dis = _dis_from_cnt(cnt_ref)
        agg = (acc_ref[0] + acc_ref[1]) * dis
        z = jnp.maximum(agg + sc_ref[...], 0.0)
        o_ref[...] = jnp.dot(z, w_ref[...],
                             preferred_element_type=jnp.float32) + b_ref[...]

    return pl.pallas_call(
        body,
        grid=(n // blk,),
        in_specs=[
            pl.BlockSpec((NC, blk, d), lambda i: (0, i, 0)),
            pl.BlockSpec((NC, blk, 16), lambda i: (0, i, 0)),
            pl.BlockSpec((blk, d), lambda i: (i, 0)),
            pl.BlockSpec((d, dout), lambda i: (0, 0)),
            pl.BlockSpec((1, dout), lambda i: (0, 0)),
        ],
        out_specs=pl.BlockSpec((blk, dout), lambda i: (i, 0)),
        out_shape=jax.ShapeDtypeStruct((n, dout), jnp.float32),
    )(acc, cnt16, sc, wl, bl)


def kernel(x, edge_index, W1, b1, W2, b2, Wl, bl):
    n = x.shape[0]
    src2 = edge_index[0].reshape(-1, EDGE_BLK)
    dst2 = edge_index[1].reshape(-1, EDGE_BLK)
    b1r = b1.reshape(1, -1)
    b2r = b2.reshape(1, -1)
    blr = bl.reshape(1, -1)

    cnt16 = _sc_degree_pass(dst2, n)
    h1 = _tc_mm(x, W1)
    hs1, sc1 = _tc_scale(cnt16, h1, b1r)

    acc1 = _sc_gather_pass(hs1, src2, dst2, n, hs1.shape[1])
    hs2, sc2 = _tc_combine_mm(acc1, cnt16, sc1, W2, b2r)

    acc2 = _sc_gather_pass(hs2, src2, dst2, n, hs2.shape[1])
    return _tc_final(acc2, cnt16, sc2, Wl, blr)
